# baseline (speedup 1.0000x reference)
import functools

import jax
import jax.numpy as jnp
from jax import lax
from jax.experimental import pallas as pl
from jax.experimental.pallas import tpu as pltpu
from jax.experimental.pallas import tpu_sc as plsc

ENT_NUM = 1000000
REL_NUM = 1000
EMB_DIM = 64
BATCH = 16384

NC = 2
NS = 16
NW = NC * NS
B_PER_W = BATCH // NW
GCHUNK = 128
NG = B_PER_W // GCHUNK
L = 16
NCH = EMB_DIM // L


def _body(heads_hbm, rels_hbm, tails_hbm, ent_hbm, rel_hbm, out_hbm,
          hidx, ridx, tidx, hrows, rrows, trows, outv, sem):
    wid = lax.axis_index("s") * NC + lax.axis_index("c")
    base = pl.multiple_of(wid * B_PER_W, B_PER_W)

    pltpu.sync_copy(heads_hbm.at[pl.ds(base, B_PER_W)], hidx)
    pltpu.sync_copy(rels_hbm.at[pl.ds(base, B_PER_W)], ridx)
    pltpu.sync_copy(tails_hbm.at[pl.ds(base, B_PER_W)], tidx)

    copies = []
    for j in range(NG):
        sl = pl.ds(j * GCHUNK, GCHUNK)
        copies.append(pltpu.async_copy(ent_hbm.at[hidx.at[sl]], hrows.at[sl], sem))
        copies.append(pltpu.async_copy(ent_hbm.at[tidx.at[sl]], trows.at[sl], sem))
        copies.append(pltpu.async_copy(rel_hbm.at[ridx.at[sl]], rrows.at[sl], sem))
    for c in copies:
        c.wait()

    lanes = lax.iota(jnp.int32, L)

    def blk(j, _):
        acc16 = jnp.zeros((L,), jnp.float32)
        for k in range(L):
            i = j * L + k
            acc = (hrows[i, pl.ds(0, L)] * rrows[i, pl.ds(0, L)]
                   * trows[i, pl.ds(0, L)])
            for c in range(1, NCH):
                acc = acc + (hrows[i, pl.ds(c * L, L)]
                             * rrows[i, pl.ds(c * L, L)]
                             * trows[i, pl.ds(c * L, L)])
            s = lax.reduce_sum(acc, axes=(0,))
            acc16 = jnp.where(lanes == k, s, acc16)
        outv[pl.ds(pl.multiple_of(j * L, L), L)] = acc16
        return 0

    lax.fori_loop(0, B_PER_W // L, blk, 0)

    pltpu.sync_copy(outv, out_hbm.at[pl.ds(base, B_PER_W)])


@jax.jit
def _distmult(heads, rels, tails, ent_embeds, rel_embeds):
    mesh = plsc.VectorSubcoreMesh(core_axis_name="c", subcore_axis_name="s")
    return pl.kernel(
        _body,
        out_type=jax.ShapeDtypeStruct((BATCH,), jnp.float32),
        mesh=mesh,
        compiler_params=pltpu.CompilerParams(
            needs_layout_passes=False, use_tc_tiling_on_sc=False),
        scratch_types=[
            pltpu.VMEM((B_PER_W,), jnp.int32),
            pltpu.VMEM((B_PER_W,), jnp.int32),
            pltpu.VMEM((B_PER_W,), jnp.int32),
            pltpu.VMEM((B_PER_W, EMB_DIM), jnp.float32),
            pltpu.VMEM((B_PER_W, EMB_DIM), jnp.float32),
            pltpu.VMEM((B_PER_W, EMB_DIM), jnp.float32),
            pltpu.VMEM((B_PER_W,), jnp.float32),
            pltpu.SemaphoreType.DMA,
        ],
    )(heads, rels, tails, ent_embeds, rel_embeds)


def kernel(heads, rels, tails, ent_embeds, rel_embeds):
    return _distmult(heads.astype(jnp.int32), rels.astype(jnp.int32),
                     tails.astype(jnp.int32), ent_embeds, rel_embeds)


# speedup vs baseline: 1.6136x; 1.6136x over previous
import functools

import jax
import jax.numpy as jnp
from jax import lax
from jax.experimental import pallas as pl
from jax.experimental.pallas import tpu as pltpu
from jax.experimental.pallas import tpu_sc as plsc

ENT_NUM = 1000000
REL_NUM = 1000
EMB_DIM = 64
BATCH = 16384

NC = 2
NS = 16
NW = NC * NS
B_PER_W = BATCH // NW
L = 16
CHUNK = L
NCHUNK = B_PER_W // CHUNK
NCH = EMB_DIM // L


def _body(heads_hbm, rels_hbm, tails_hbm, ent_hbm, rel_hbm, out_hbm,
          hidx, ridx, tidx, hbuf, rbuf, tbuf, outv, sem):
    wid = lax.axis_index("s") * NC + lax.axis_index("c")
    base = pl.multiple_of(wid * B_PER_W, B_PER_W)

    pltpu.sync_copy(heads_hbm.at[pl.ds(base, B_PER_W)], hidx)
    pltpu.sync_copy(rels_hbm.at[pl.ds(base, B_PER_W)], ridx)
    pltpu.sync_copy(tails_hbm.at[pl.ds(base, B_PER_W)], tidx)

    lanes = lax.iota(jnp.int32, L)

    def chunk(c, _):
        cbase = c * CHUNK
        gsl = pl.ds(pl.multiple_of(cbase, CHUNK), CHUNK)
        hv = hidx[gsl]
        rv = ridx[gsl]
        tv = tidx[gsl]
        cps = []
        for k in range(L):
            cps.append(pltpu.async_copy(ent_hbm.at[hv[k]], hbuf.at[k], sem))
            cps.append(pltpu.async_copy(ent_hbm.at[tv[k]], tbuf.at[k], sem))
            cps.append(pltpu.async_copy(rel_hbm.at[rv[k]], rbuf.at[k], sem))
        for cp in cps:
            cp.wait()
        acc16 = jnp.zeros((L,), jnp.float32)
        for k in range(L):
            acc = (hbuf[k, pl.ds(0, L)] * rbuf[k, pl.ds(0, L)]
                   * tbuf[k, pl.ds(0, L)])
            for cc in range(1, NCH):
                acc = acc + (hbuf[k, pl.ds(cc * L, L)]
                             * rbuf[k, pl.ds(cc * L, L)]
                             * tbuf[k, pl.ds(cc * L, L)])
            s = lax.reduce_sum(acc, axes=(0,))
            acc16 = jnp.where(lanes == k, s, acc16)
        outv[gsl] = acc16
        return 0

    lax.fori_loop(0, NCHUNK, chunk, 0)

    pltpu.sync_copy(outv, out_hbm.at[pl.ds(base, B_PER_W)])


@jax.jit
def _distmult(heads, rels, tails, ent_embeds, rel_embeds):
    mesh = plsc.VectorSubcoreMesh(core_axis_name="c", subcore_axis_name="s")
    return pl.kernel(
        _body,
        out_type=jax.ShapeDtypeStruct((BATCH,), jnp.float32),
        mesh=mesh,
        compiler_params=pltpu.CompilerParams(
            needs_layout_passes=False, use_tc_tiling_on_sc=True),
        scratch_types=[
            pltpu.VMEM((B_PER_W,), jnp.int32),
            pltpu.VMEM((B_PER_W,), jnp.int32),
            pltpu.VMEM((B_PER_W,), jnp.int32),
            pltpu.VMEM((CHUNK, EMB_DIM), jnp.float32),
            pltpu.VMEM((CHUNK, EMB_DIM), jnp.float32),
            pltpu.VMEM((CHUNK, EMB_DIM), jnp.float32),
            pltpu.VMEM((B_PER_W,), jnp.float32),
            pltpu.SemaphoreType.DMA,
        ],
    )(heads, rels, tails, ent_embeds, rel_embeds)


def kernel(heads, rels, tails, ent_embeds, rel_embeds):
    return _distmult(heads.astype(jnp.int32), rels.astype(jnp.int32),
                     tails.astype(jnp.int32), ent_embeds, rel_embeds)
